# Initial kernel scaffold; baseline (speedup 1.0000x reference)
import jax
import jax.numpy as jnp
from jax.experimental import pallas as pl


def kernel(x, emb_weight):
    raise NotImplementedError("write your pallas kernel here")


# speedup vs baseline: 1.5912x; 1.5912x over previous
import jax
import jax.numpy as jnp
from jax.experimental import pallas as pl

SEQ_BLOCK = 512


def _add_kernel(x_ref, emb_ref, out_ref):
    out_ref[...] = x_ref[...] + emb_ref[...]


def kernel(x, emb_weight):
    batch, seq_len, d_model = x.shape
    pos_emb = emb_weight[:seq_len]
    n_seq_blocks = seq_len // SEQ_BLOCK
    return pl.pallas_call(
        _add_kernel,
        grid=(n_seq_blocks, batch),
        in_specs=[
            pl.BlockSpec((1, SEQ_BLOCK, d_model), lambda i, b: (b, i, 0)),
            pl.BlockSpec((1, SEQ_BLOCK, d_model), lambda i, b: (0, i, 0)),
        ],
        out_specs=pl.BlockSpec((1, SEQ_BLOCK, d_model), lambda i, b: (b, i, 0)),
        out_shape=jax.ShapeDtypeStruct(x.shape, x.dtype),
    )(x, pos_emb[None])
